# baseline (speedup 1.0000x reference)
import functools

import jax
import jax.numpy as jnp
from jax.experimental import pallas as pl
from jax.experimental.pallas import tpu as pltpu

_BETA = 0.25


def _vq_body(nc, nk, tc, z_ref, cb_ref, cn_ref, zq_ref, idx_ref, loss_ref,
             cbb_ref, cbm2_ref, arows_ref):
    c = pl.program_id(0)
    cnorm = cn_ref[...]

    @pl.when(c == 0)
    def _():
        cb0 = cb_ref[...]
        cbb_ref[...] = cb0.astype(jnp.bfloat16)
        cbm2_ref[...] = cb0 * -2.0
        kr = jax.lax.broadcasted_iota(jnp.int32, (1, nk), 1)
        arows_ref[...] = jnp.concatenate(
            [(kr // 32).astype(jnp.float32),
             (kr % 32).astype(jnp.float32)], axis=0).astype(jnp.bfloat16)

    cbb = cbb_ref[...]
    cbm2 = cbm2_ref[...]
    arows = arows_ref[...]
    liota = jax.lax.broadcasted_iota(jnp.int32, (tc, nk), 1)

    z = z_ref[...]
    s = jax.lax.dot_general(
        z, cbm2, (((1,), (1,)), ((), ())),
        preferred_element_type=jnp.float32)
    znorm = jnp.sum(z * z, axis=1, keepdims=True)
    dist = (znorm + s) + cnorm

    m = jnp.min(dist, axis=1, keepdims=True)
    idxc = jnp.min(jnp.where(dist == m, liota, nk),
                   axis=1, keepdims=True)
    onehot = (liota == idxc).astype(jnp.bfloat16)

    hilo = jax.lax.dot_general(
        arows, onehot, (((1,), (1,)), ((), ())),
        preferred_element_type=jnp.float32)
    idxrow = (32.0 * hilo[0:1] + hilo[1:2]).astype(jnp.int32)
    idx_ref[:, pl.ds(c * tc, tc)] = idxrow

    zq = jax.lax.dot_general(
        onehot, cbb, (((1,), (0,)), ((), ())),
        preferred_element_type=jnp.float32)
    d = zq - z
    zq_ref[...] = z + d
    part = jnp.sum(d * d, keepdims=True)

    @pl.when(c == 0)
    def _():
        loss_ref[...] = jnp.zeros_like(loss_ref)

    acc = loss_ref[...] + part

    @pl.when(c != nc - 1)
    def _():
        loss_ref[...] = acc

    @pl.when(c == nc - 1)
    def _():
        mean = acc / (nc * tc * z_ref.shape[1])
        loss_ref[...] = _BETA * mean + mean


def kernel(z, codebook):
    B, C, H, W = z.shape
    K = codebook.shape[0]
    NT = B * H * W
    TC = 512
    NC = NT // TC
    zf = jnp.transpose(z, (0, 2, 3, 1)).reshape(NT, C)
    cn = jnp.sum(codebook ** 2, axis=1)[None, :]

    zqf, idx2, loss11 = pl.pallas_call(
        functools.partial(_vq_body, NC, K, TC),
        grid=(NC,),
        in_specs=[
            pl.BlockSpec((TC, C), lambda i: (i, 0)),
            pl.BlockSpec((K, C), lambda i: (0, 0)),
            pl.BlockSpec((1, K), lambda i: (0, 0)),
        ],
        out_specs=[
            pl.BlockSpec((TC, C), lambda i: (i, 0)),
            pl.BlockSpec((1, NT), lambda i: (0, 0)),
            pl.BlockSpec((1, 1), lambda i: (0, 0)),
        ],
        out_shape=[
            jax.ShapeDtypeStruct((NT, C), jnp.float32),
            jax.ShapeDtypeStruct((1, NT), jnp.int32),
            jax.ShapeDtypeStruct((1, 1), jnp.float32),
        ],
        scratch_shapes=[
            pltpu.VMEM((K, C), jnp.bfloat16),
            pltpu.VMEM((K, C), jnp.float32),
            pltpu.VMEM((2, K), jnp.bfloat16),
        ],
    )(zf, codebook, cn)

    zq = jnp.transpose(zqf.reshape(B, H, W, C), (0, 3, 1, 2))
    idx = idx2.reshape(-1)
    loss = loss11.reshape(())
    return zq, idx, loss
